# baseline (speedup 1.0000x reference)
import functools

import jax
import jax.numpy as jnp
from jax import lax
from jax.experimental import pallas as pl
from jax.experimental.pallas import tpu as pltpu
from jax.experimental.pallas import tpu_sc as plsc

EPS = 1e-5
LANES = 16
NC = 2
NS = 16
NW = NC * NS


def _sc_mesh():
    return plsc.VectorSubcoreMesh(core_axis_name="c", subcore_axis_name="s")


_SC_PARAMS = pltpu.CompilerParams(needs_layout_passes=False)


def _wid():
    return lax.axis_index("c") * NS + lax.axis_index("s")


def _make_deg(n, e):
    epw = e // NW
    chunk = 2000
    ngrp = chunk // LANES

    @functools.partial(
        pl.kernel,
        out_type=jax.ShapeDtypeStruct((NW, n), jnp.float32),
        mesh=_sc_mesh(),
        compiler_params=_SC_PARAMS,
        scratch_types=[
            pltpu.VMEM((n,), jnp.float32),
            pltpu.VMEM((epw,), jnp.int32),
            pltpu.VMEM((epw,), jnp.float32),
        ],
    )
    def deg(col_hbm, ew_hbm, out_hbm, acc, colv, ewv):
        w = _wid()
        zero = jnp.zeros((LANES,), jnp.float32)

        def zbody(i, _):
            acc[pl.ds(i * LANES, LANES)] = zero
            return 0

        lax.fori_loop(0, n // LANES, zbody, 0)
        pltpu.sync_copy(col_hbm.at[pl.ds(w * epw, epw)], colv)
        pltpu.sync_copy(ew_hbm.at[pl.ds(w * epw, epw)], ewv)

        @plsc.parallel_loop(0, epw // LANES, 1, unroll=5)
        def grp(g):
            c16 = colv[pl.ds(g * LANES, LANES)]
            w16 = ewv[pl.ds(g * LANES, LANES)]
            plsc.addupdate_scatter(acc, [c16], w16)

        pltpu.sync_copy(acc, out_hbm.at[w])

    return deg


def _make_edge_scatter(n, e, h):
    cpw = h // NW
    chunk = 1600
    ngrp = chunk // LANES
    nchunk = e // chunk
    unroll = 10

    @functools.partial(
        pl.kernel,
        out_type=jax.ShapeDtypeStruct((h * n,), jnp.float32),
        mesh=_sc_mesh(),
        compiler_params=_SC_PARAMS,
        scratch_types=(
            [pltpu.VMEM((n,), jnp.float32)] * cpw
            + [pltpu.VMEM((n,), jnp.float32)] * cpw
            + [
                pltpu.VMEM((chunk,), jnp.int32),
                pltpu.VMEM((chunk,), jnp.int32),
                pltpu.VMEM((chunk,), jnp.float32),
                pltpu.VMEM((chunk,), jnp.int32),
                pltpu.VMEM((chunk,), jnp.int32),
                pltpu.VMEM((chunk,), jnp.float32),
                pltpu.SemaphoreType.DMA,
                pltpu.SemaphoreType.DMA,
            ]
        ),
    )
    def scat(ut_hbm, row_hbm, col_hbm, ew_hbm, out_hbm,
             u0, u1, u2, u3, a0, a1, a2, a3,
             r0, c0, w0, r1, c1, w1_, s0, s1):
        ubufs = (u0, u1, u2, u3)
        accs = (a0, a1, a2, a3)
        bufs = ((r0, c0, w0, s0), (r1, c1, w1_, s1))
        w = _wid()
        rot = w * (nchunk // NW)
        for ch in range(cpw):
            pltpu.sync_copy(ut_hbm.at[pl.ds((w * cpw + ch) * n, n)], ubufs[ch])
        zero = jnp.zeros((LANES,), jnp.float32)

        def zbody(i, _):
            for ch in range(cpw):
                accs[ch][pl.ds(i * LANES, LANES)] = zero
            return 0

        lax.fori_loop(0, n // LANES, zbody, 0)

        def start(cid, b):
            base = lax.rem(cid + rot, nchunk) * chunk
            rv, cv, wv, sem = bufs[b]
            pltpu.async_copy(row_hbm.at[pl.ds(base, chunk)], rv, sem)
            pltpu.async_copy(col_hbm.at[pl.ds(base, chunk)], cv, sem)
            pltpu.async_copy(ew_hbm.at[pl.ds(base, chunk)], wv, sem)

        def wait(b):
            rv, cv, wv, sem = bufs[b]
            pltpu.make_async_copy(row_hbm.at[pl.ds(0, chunk)], rv, sem).wait()
            pltpu.make_async_copy(col_hbm.at[pl.ds(0, chunk)], cv, sem).wait()
            pltpu.make_async_copy(ew_hbm.at[pl.ds(0, chunk)], wv, sem).wait()

        start(0, 0)

        def outer(k, _):
            for b in range(2):
                cid = k * 2 + b

                @pl.when(cid + 1 < nchunk)
                def _():
                    start(cid + 1, 1 - b)

                wait(b)
                rv, cv, wv, _sem = bufs[b]

                @plsc.parallel_loop(0, ngrp, 1, unroll=unroll)
                def grp(g):
                    gb = g * LANES
                    r16 = rv[pl.ds(gb, LANES)]
                    c16 = cv[pl.ds(gb, LANES)]
                    w16 = wv[pl.ds(gb, LANES)]
                    for ch in range(cpw):
                        vals = plsc.load_gather(ubufs[ch], [r16]) * w16
                        plsc.addupdate_scatter(accs[ch], [c16], vals)
            return 0

        lax.fori_loop(0, nchunk // 2, outer, 0)
        for ch in range(cpw):
            pltpu.sync_copy(accs[ch], out_hbm.at[pl.ds((w * cpw + ch) * n, n)])

    return scat


def _make_edge_scalar(n, e):
    epw = e // NW
    chunk = 2000
    ngrp = chunk // LANES

    @functools.partial(
        pl.kernel,
        out_type=jax.ShapeDtypeStruct((NW, n), jnp.float32),
        mesh=_sc_mesh(),
        compiler_params=_SC_PARAMS,
        scratch_types=[
            pltpu.VMEM((n,), jnp.float32),
            pltpu.VMEM((n,), jnp.float32),
            pltpu.VMEM((epw,), jnp.int32),
            pltpu.VMEM((epw,), jnp.int32),
            pltpu.VMEM((epw,), jnp.float32),
        ],
    )
    def scat1(u_hbm, row_hbm, col_hbm, ew_hbm, out_hbm, ubuf, acc, rowv, colv, ewv):
        w = _wid()
        pltpu.sync_copy(u_hbm, ubuf)
        zero = jnp.zeros((LANES,), jnp.float32)

        def zbody(i, _):
            acc[pl.ds(i * LANES, LANES)] = zero
            return 0

        lax.fori_loop(0, n // LANES, zbody, 0)
        pltpu.sync_copy(row_hbm.at[pl.ds(w * epw, epw)], rowv)
        pltpu.sync_copy(col_hbm.at[pl.ds(w * epw, epw)], colv)
        pltpu.sync_copy(ew_hbm.at[pl.ds(w * epw, epw)], ewv)

        @plsc.parallel_loop(0, epw // LANES, 1, unroll=5)
        def grp(g):
            r16 = rowv[pl.ds(g * LANES, LANES)]
            c16 = colv[pl.ds(g * LANES, LANES)]
            w16 = ewv[pl.ds(g * LANES, LANES)]
            vals = plsc.load_gather(ubuf, [r16]) * w16
            plsc.addupdate_scatter(acc, [c16], vals)

        pltpu.sync_copy(acc, out_hbm.at[w])

    return scat1


def _tc_first(x, w1, degp):
    n = x.shape[0]
    h = w1.shape[1]

    def body(x_ref, w_ref, degp_ref, u_ref, dinv_ref):
        deg = 1.0 + jnp.sum(degp_ref[...], axis=0, keepdims=True)
        dinv = jnp.where(deg > 0, lax.rsqrt(deg), 0.0)
        xw = lax.dot_general(w_ref[...], x_ref[...], (((0,), (1,)), ((), ())),
                             preferred_element_type=jnp.float32)
        u_ref[...] = xw * dinv
        dinv_ref[...] = dinv

    return pl.pallas_call(
        body,
        out_shape=[jax.ShapeDtypeStruct((h, n), jnp.float32),
                   jax.ShapeDtypeStruct((1, n), jnp.float32)],
    )(x, w1, degp)


def _tc_mid(sT, uT, dinv, b, g, be, wn):
    h = sT.shape[0]
    n = sT.shape[1]
    hn = wn.shape[1]

    def body(s_ref, u_ref, dinv_ref, b_ref, g_ref, be_ref, w_ref, o_ref):
        dv = dinv_ref[...]
        z = dv * (s_ref[...] + u_ref[...]) + b_ref[...]
        m = jnp.mean(z, axis=1, keepdims=True)
        zc = z - m
        v = jnp.mean(zc * zc, axis=1, keepdims=True)
        hh = zc * lax.rsqrt(v + EPS) * g_ref[...] + be_ref[...]
        hh = jnp.maximum(hh, 0.0)
        o_ref[...] = lax.dot_general(w_ref[...], hh, (((0,), (0,)), ((), ())),
                                     preferred_element_type=jnp.float32) * dv

    return pl.pallas_call(
        body,
        out_shape=jax.ShapeDtypeStruct((hn, n), jnp.float32),
    )(sT, uT, dinv, b, g, be, wn)


def _tc_final(s3p, u3, dinv, b3):
    n = u3.shape[1]

    def body(s_ref, u_ref, dinv_ref, b_ref, o_ref):
        s = jnp.sum(s_ref[...], axis=0, keepdims=True)
        o_ref[...] = dinv_ref[...] * (s + u_ref[...]) + b_ref[...]

    return pl.pallas_call(
        body,
        out_shape=jax.ShapeDtypeStruct((1, n), jnp.float32),
    )(s3p, u3, dinv, b3)


def kernel(x, edge_index, edge_attr, W1, b1, g1, be1, W2, b2, g2, be2, W3, b3):
    n = x.shape[0]
    e = edge_attr.shape[0]
    h = W1.shape[1]

    row = edge_index[0].astype(jnp.int32)
    col = edge_index[1].astype(jnp.int32)
    ew = edge_attr.astype(jnp.float32)

    degp = _make_deg(n, e)(col, ew)
    u1T, dinv = _tc_first(x, W1, degp)
    scat = _make_edge_scatter(n, e, h)
    s1T = scat(u1T.reshape(h * n), row, col, ew).reshape(h, n)
    u2T = _tc_mid(s1T, u1T, dinv, b1[:, None], g1[:, None], be1[:, None], W2)
    s2T = scat(u2T.reshape(h * n), row, col, ew).reshape(h, n)
    u3 = _tc_mid(s2T, u2T, dinv, b2[:, None], g2[:, None], be2[:, None], W3)
    s3p = _make_edge_scalar(n, e)(u3.reshape(n), row, col, ew)
    out = _tc_final(s3p, u3, dinv, b3[:, None])
    return out.reshape(n, 1)


# speedup vs baseline: 1.0192x; 1.0192x over previous
import functools

import jax
import jax.numpy as jnp
from jax import lax
from jax.experimental import pallas as pl
from jax.experimental.pallas import tpu as pltpu
from jax.experimental.pallas import tpu_sc as plsc

EPS = 1e-5
LANES = 16
NC = 2
NS = 16
NW = NC * NS


def _sc_mesh():
    return plsc.VectorSubcoreMesh(core_axis_name="c", subcore_axis_name="s")


_SC_PARAMS = pltpu.CompilerParams(needs_layout_passes=False)


def _wid():
    return lax.axis_index("c") * NS + lax.axis_index("s")


def _make_deg(n, e):
    epw = e // NW
    chunk = 2000
    ngrp = chunk // LANES

    @functools.partial(
        pl.kernel,
        out_type=jax.ShapeDtypeStruct((NW, n), jnp.float32),
        mesh=_sc_mesh(),
        compiler_params=_SC_PARAMS,
        scratch_types=[
            pltpu.VMEM((n,), jnp.float32),
            pltpu.VMEM((epw,), jnp.int32),
            pltpu.VMEM((epw,), jnp.float32),
        ],
    )
    def deg(col_hbm, ew_hbm, out_hbm, acc, colv, ewv):
        w = _wid()
        zero = jnp.zeros((LANES,), jnp.float32)

        def zbody(i, _):
            acc[pl.ds(i * LANES, LANES)] = zero
            return 0

        lax.fori_loop(0, n // LANES, zbody, 0)
        pltpu.sync_copy(col_hbm.at[pl.ds(w * epw, epw)], colv)
        pltpu.sync_copy(ew_hbm.at[pl.ds(w * epw, epw)], ewv)

        @plsc.parallel_loop(0, epw // LANES, 1, unroll=5)
        def grp(g):
            c16 = colv[pl.ds(g * LANES, LANES)]
            w16 = ewv[pl.ds(g * LANES, LANES)]
            plsc.addupdate_scatter(acc, [c16], w16)

        pltpu.sync_copy(acc, out_hbm.at[w])

    return deg


def _make_edge_scatter(n, e, h):
    cpw = h // NW
    chunk = 2000
    ngrp = chunk // LANES
    nchunk = e // chunk
    unroll = 5

    @functools.partial(
        pl.kernel,
        out_type=jax.ShapeDtypeStruct((h * n,), jnp.float32),
        mesh=_sc_mesh(),
        compiler_params=_SC_PARAMS,
        scratch_types=(
            [pltpu.VMEM((n,), jnp.float32)] * cpw
            + [pltpu.VMEM((n,), jnp.float32)] * cpw
            + [
                pltpu.VMEM((chunk,), jnp.int32),
                pltpu.VMEM((chunk,), jnp.int32),
                pltpu.VMEM((chunk,), jnp.float32),
                pltpu.VMEM((chunk,), jnp.int32),
                pltpu.VMEM((chunk,), jnp.int32),
                pltpu.VMEM((chunk,), jnp.float32),
                pltpu.SemaphoreType.DMA,
                pltpu.SemaphoreType.DMA,
            ]
        ),
    )
    def scat(ut_hbm, row_hbm, col_hbm, ew_hbm, out_hbm,
             u0, u1, u2, u3, a0, a1, a2, a3,
             r0, c0, w0, r1, c1, w1_, s0, s1):
        ubufs = (u0, u1, u2, u3)
        accs = (a0, a1, a2, a3)
        bufs = ((r0, c0, w0, s0), (r1, c1, w1_, s1))
        w = _wid()
        rot = w * (nchunk // NW)
        for ch in range(cpw):
            pltpu.sync_copy(ut_hbm.at[pl.ds((w * cpw + ch) * n, n)], ubufs[ch])
        zero = jnp.zeros((LANES,), jnp.float32)

        def zbody(i, _):
            for ch in range(cpw):
                accs[ch][pl.ds(i * LANES, LANES)] = zero
            return 0

        lax.fori_loop(0, n // LANES, zbody, 0)

        def start(cid, b):
            base = lax.rem(cid + rot, nchunk) * chunk
            rv, cv, wv, sem = bufs[b]
            pltpu.async_copy(row_hbm.at[pl.ds(base, chunk)], rv, sem)
            pltpu.async_copy(col_hbm.at[pl.ds(base, chunk)], cv, sem)
            pltpu.async_copy(ew_hbm.at[pl.ds(base, chunk)], wv, sem)

        def wait(b):
            rv, cv, wv, sem = bufs[b]
            pltpu.make_async_copy(row_hbm.at[pl.ds(0, chunk)], rv, sem).wait()
            pltpu.make_async_copy(col_hbm.at[pl.ds(0, chunk)], cv, sem).wait()
            pltpu.make_async_copy(ew_hbm.at[pl.ds(0, chunk)], wv, sem).wait()

        start(0, 0)

        def outer(k, _):
            for b in range(2):
                cid = k * 2 + b

                @pl.when(cid + 1 < nchunk)
                def _():
                    start(cid + 1, 1 - b)

                wait(b)
                rv, cv, wv, _sem = bufs[b]

                @plsc.parallel_loop(0, ngrp, 1, unroll=unroll)
                def grp(g):
                    gb = g * LANES
                    r16 = rv[pl.ds(gb, LANES)]
                    c16 = cv[pl.ds(gb, LANES)]
                    w16 = wv[pl.ds(gb, LANES)]
                    for ch in range(cpw):
                        vals = plsc.load_gather(ubufs[ch], [r16]) * w16
                        plsc.addupdate_scatter(accs[ch], [c16], vals)
            return 0

        lax.fori_loop(0, nchunk // 2, outer, 0)
        for ch in range(cpw):
            pltpu.sync_copy(accs[ch], out_hbm.at[pl.ds((w * cpw + ch) * n, n)])

    return scat


def _make_edge_scalar(n, e):
    epw = e // NW
    chunk = 2000
    ngrp = chunk // LANES

    @functools.partial(
        pl.kernel,
        out_type=jax.ShapeDtypeStruct((NW, n), jnp.float32),
        mesh=_sc_mesh(),
        compiler_params=_SC_PARAMS,
        scratch_types=[
            pltpu.VMEM((n,), jnp.float32),
            pltpu.VMEM((n,), jnp.float32),
            pltpu.VMEM((epw,), jnp.int32),
            pltpu.VMEM((epw,), jnp.int32),
            pltpu.VMEM((epw,), jnp.float32),
        ],
    )
    def scat1(u_hbm, row_hbm, col_hbm, ew_hbm, out_hbm, ubuf, acc, rowv, colv, ewv):
        w = _wid()
        pltpu.sync_copy(u_hbm, ubuf)
        zero = jnp.zeros((LANES,), jnp.float32)

        def zbody(i, _):
            acc[pl.ds(i * LANES, LANES)] = zero
            return 0

        lax.fori_loop(0, n // LANES, zbody, 0)
        pltpu.sync_copy(row_hbm.at[pl.ds(w * epw, epw)], rowv)
        pltpu.sync_copy(col_hbm.at[pl.ds(w * epw, epw)], colv)
        pltpu.sync_copy(ew_hbm.at[pl.ds(w * epw, epw)], ewv)

        @plsc.parallel_loop(0, epw // LANES, 1, unroll=5)
        def grp(g):
            r16 = rowv[pl.ds(g * LANES, LANES)]
            c16 = colv[pl.ds(g * LANES, LANES)]
            w16 = ewv[pl.ds(g * LANES, LANES)]
            vals = plsc.load_gather(ubuf, [r16]) * w16
            plsc.addupdate_scatter(acc, [c16], vals)

        pltpu.sync_copy(acc, out_hbm.at[w])

    return scat1


def _tc_first(x, w1, degp):
    n = x.shape[0]
    h = w1.shape[1]

    def body(x_ref, w_ref, degp_ref, u_ref, dinv_ref):
        deg = 1.0 + jnp.sum(degp_ref[...], axis=0, keepdims=True)
        dinv = jnp.where(deg > 0, lax.rsqrt(deg), 0.0)
        xw = lax.dot_general(w_ref[...], x_ref[...], (((0,), (1,)), ((), ())),
                             preferred_element_type=jnp.float32)
        u_ref[...] = xw * dinv
        dinv_ref[...] = dinv

    return pl.pallas_call(
        body,
        out_shape=[jax.ShapeDtypeStruct((h, n), jnp.float32),
                   jax.ShapeDtypeStruct((1, n), jnp.float32)],
    )(x, w1, degp)


def _tc_mid(sT, uT, dinv, b, g, be, wn):
    h = sT.shape[0]
    n = sT.shape[1]
    hn = wn.shape[1]

    def body(s_ref, u_ref, dinv_ref, b_ref, g_ref, be_ref, w_ref, o_ref):
        dv = dinv_ref[...]
        z = dv * (s_ref[...] + u_ref[...]) + b_ref[...]
        m = jnp.mean(z, axis=1, keepdims=True)
        zc = z - m
        v = jnp.mean(zc * zc, axis=1, keepdims=True)
        hh = zc * lax.rsqrt(v + EPS) * g_ref[...] + be_ref[...]
        hh = jnp.maximum(hh, 0.0)
        o_ref[...] = lax.dot_general(w_ref[...], hh, (((0,), (0,)), ((), ())),
                                     preferred_element_type=jnp.float32) * dv

    return pl.pallas_call(
        body,
        out_shape=jax.ShapeDtypeStruct((hn, n), jnp.float32),
    )(sT, uT, dinv, b, g, be, wn)


def _tc_final(s3p, u3, dinv, b3):
    n = u3.shape[1]

    def body(s_ref, u_ref, dinv_ref, b_ref, o_ref):
        s = jnp.sum(s_ref[...], axis=0, keepdims=True)
        o_ref[...] = dinv_ref[...] * (s + u_ref[...]) + b_ref[...]

    return pl.pallas_call(
        body,
        out_shape=jax.ShapeDtypeStruct((1, n), jnp.float32),
    )(s3p, u3, dinv, b3)


def kernel(x, edge_index, edge_attr, W1, b1, g1, be1, W2, b2, g2, be2, W3, b3):
    n = x.shape[0]
    e = edge_attr.shape[0]
    h = W1.shape[1]

    row = edge_index[0].astype(jnp.int32)
    col = edge_index[1].astype(jnp.int32)
    ew = edge_attr.astype(jnp.float32)

    degp = _make_deg(n, e)(col, ew)
    u1T, dinv = _tc_first(x, W1, degp)
    scat = _make_edge_scatter(n, e, h)
    s1T = scat(u1T.reshape(h * n), row, col, ew).reshape(h, n)
    u2T = _tc_mid(s1T, u1T, dinv, b1[:, None], g1[:, None], be1[:, None], W2)
    s2T = scat(u2T.reshape(h * n), row, col, ew).reshape(h, n)
    u3 = _tc_mid(s2T, u2T, dinv, b2[:, None], g2[:, None], be2[:, None], W3)
    s3p = _make_edge_scalar(n, e)(u3.reshape(n), row, col, ew)
    out = _tc_final(s3p, u3, dinv, b3[:, None])
    return out.reshape(n, 1)


# speedup vs baseline: 1.0211x; 1.0019x over previous
import functools

import jax
import jax.numpy as jnp
from jax import lax
from jax.experimental import pallas as pl
from jax.experimental.pallas import tpu as pltpu
from jax.experimental.pallas import tpu_sc as plsc

EPS = 1e-5
LANES = 16
NC = 2
NS = 16
NW = NC * NS


def _sc_mesh():
    return plsc.VectorSubcoreMesh(core_axis_name="c", subcore_axis_name="s")


_SC_PARAMS = pltpu.CompilerParams(needs_layout_passes=False)


def _wid():
    return lax.axis_index("c") * NS + lax.axis_index("s")


def _make_deg(n, e):
    epw = e // NW
    chunk = 2000
    ngrp = chunk // LANES

    @functools.partial(
        pl.kernel,
        out_type=jax.ShapeDtypeStruct((NW, n), jnp.float32),
        mesh=_sc_mesh(),
        compiler_params=_SC_PARAMS,
        scratch_types=[
            pltpu.VMEM((n,), jnp.float32),
            pltpu.VMEM((epw,), jnp.int32),
            pltpu.VMEM((epw,), jnp.float32),
        ],
    )
    def deg(col_hbm, ew_hbm, out_hbm, acc, colv, ewv):
        w = _wid()
        zero = jnp.zeros((LANES,), jnp.float32)

        def zbody(i, _):
            acc[pl.ds(i * LANES, LANES)] = zero
            return 0

        lax.fori_loop(0, n // LANES, zbody, 0)
        pltpu.sync_copy(col_hbm.at[pl.ds(w * epw, epw)], colv)
        pltpu.sync_copy(ew_hbm.at[pl.ds(w * epw, epw)], ewv)

        @plsc.parallel_loop(0, epw // LANES, 1, unroll=5)
        def grp(g):
            c16 = colv[pl.ds(g * LANES, LANES)]
            w16 = ewv[pl.ds(g * LANES, LANES)]
            plsc.addupdate_scatter(acc, [c16], w16)

        pltpu.sync_copy(acc, out_hbm.at[w])

    return deg


def _make_edge_scatter(n, e, h):
    cpw = h // NW
    chunk = 4000
    ngrp = chunk // LANES
    nchunk = e // chunk
    unroll = 5

    @functools.partial(
        pl.kernel,
        out_type=jax.ShapeDtypeStruct((h * n,), jnp.float32),
        mesh=_sc_mesh(),
        compiler_params=_SC_PARAMS,
        scratch_types=(
            [pltpu.VMEM((n,), jnp.float32)] * cpw
            + [pltpu.VMEM((n,), jnp.float32)] * cpw
            + [
                pltpu.VMEM((chunk,), jnp.int32),
                pltpu.VMEM((chunk,), jnp.int32),
                pltpu.VMEM((chunk,), jnp.float32),
                pltpu.VMEM((chunk,), jnp.int32),
                pltpu.VMEM((chunk,), jnp.int32),
                pltpu.VMEM((chunk,), jnp.float32),
                pltpu.SemaphoreType.DMA,
                pltpu.SemaphoreType.DMA,
            ]
        ),
    )
    def scat(ut_hbm, row_hbm, col_hbm, ew_hbm, out_hbm,
             u0, u1, u2, u3, a0, a1, a2, a3,
             r0, c0, w0, r1, c1, w1_, s0, s1):
        ubufs = (u0, u1, u2, u3)
        accs = (a0, a1, a2, a3)
        bufs = ((r0, c0, w0, s0), (r1, c1, w1_, s1))
        w = _wid()
        rot = w * (nchunk // NW)
        for ch in range(cpw):
            pltpu.sync_copy(ut_hbm.at[pl.ds((w * cpw + ch) * n, n)], ubufs[ch])
        zero = jnp.zeros((LANES,), jnp.float32)

        def zbody(i, _):
            for ch in range(cpw):
                accs[ch][pl.ds(i * LANES, LANES)] = zero
            return 0

        lax.fori_loop(0, n // LANES, zbody, 0)

        def start(cid, b):
            base = lax.rem(cid + rot, nchunk) * chunk
            rv, cv, wv, sem = bufs[b]
            pltpu.async_copy(row_hbm.at[pl.ds(base, chunk)], rv, sem)
            pltpu.async_copy(col_hbm.at[pl.ds(base, chunk)], cv, sem)
            pltpu.async_copy(ew_hbm.at[pl.ds(base, chunk)], wv, sem)

        def wait(b):
            rv, cv, wv, sem = bufs[b]
            pltpu.make_async_copy(row_hbm.at[pl.ds(0, chunk)], rv, sem).wait()
            pltpu.make_async_copy(col_hbm.at[pl.ds(0, chunk)], cv, sem).wait()
            pltpu.make_async_copy(ew_hbm.at[pl.ds(0, chunk)], wv, sem).wait()

        start(0, 0)

        def outer(k, _):
            for b in range(2):
                cid = k * 2 + b

                @pl.when(cid + 1 < nchunk)
                def _():
                    start(cid + 1, 1 - b)

                wait(b)
                rv, cv, wv, _sem = bufs[b]

                @plsc.parallel_loop(0, ngrp, 1, unroll=unroll)
                def grp(g):
                    gb = g * LANES
                    r16 = rv[pl.ds(gb, LANES)]
                    c16 = cv[pl.ds(gb, LANES)]
                    w16 = wv[pl.ds(gb, LANES)]
                    for ch in range(cpw):
                        vals = plsc.load_gather(ubufs[ch], [r16]) * w16
                        plsc.addupdate_scatter(accs[ch], [c16], vals)
            return 0

        lax.fori_loop(0, nchunk // 2, outer, 0)
        for ch in range(cpw):
            pltpu.sync_copy(accs[ch], out_hbm.at[pl.ds((w * cpw + ch) * n, n)])

    return scat


def _make_edge_scalar(n, e):
    epw = e // NW
    chunk = 2000
    ngrp = chunk // LANES

    @functools.partial(
        pl.kernel,
        out_type=jax.ShapeDtypeStruct((NW, n), jnp.float32),
        mesh=_sc_mesh(),
        compiler_params=_SC_PARAMS,
        scratch_types=[
            pltpu.VMEM((n,), jnp.float32),
            pltpu.VMEM((n,), jnp.float32),
            pltpu.VMEM((epw,), jnp.int32),
            pltpu.VMEM((epw,), jnp.int32),
            pltpu.VMEM((epw,), jnp.float32),
        ],
    )
    def scat1(u_hbm, row_hbm, col_hbm, ew_hbm, out_hbm, ubuf, acc, rowv, colv, ewv):
        w = _wid()
        pltpu.sync_copy(u_hbm, ubuf)
        zero = jnp.zeros((LANES,), jnp.float32)

        def zbody(i, _):
            acc[pl.ds(i * LANES, LANES)] = zero
            return 0

        lax.fori_loop(0, n // LANES, zbody, 0)
        pltpu.sync_copy(row_hbm.at[pl.ds(w * epw, epw)], rowv)
        pltpu.sync_copy(col_hbm.at[pl.ds(w * epw, epw)], colv)
        pltpu.sync_copy(ew_hbm.at[pl.ds(w * epw, epw)], ewv)

        @plsc.parallel_loop(0, epw // LANES, 1, unroll=5)
        def grp(g):
            r16 = rowv[pl.ds(g * LANES, LANES)]
            c16 = colv[pl.ds(g * LANES, LANES)]
            w16 = ewv[pl.ds(g * LANES, LANES)]
            vals = plsc.load_gather(ubuf, [r16]) * w16
            plsc.addupdate_scatter(acc, [c16], vals)

        pltpu.sync_copy(acc, out_hbm.at[w])

    return scat1


def _tc_first(x, w1, degp):
    n = x.shape[0]
    h = w1.shape[1]

    def body(x_ref, w_ref, degp_ref, u_ref, dinv_ref):
        deg = 1.0 + jnp.sum(degp_ref[...], axis=0, keepdims=True)
        dinv = jnp.where(deg > 0, lax.rsqrt(deg), 0.0)
        xw = lax.dot_general(w_ref[...], x_ref[...], (((0,), (1,)), ((), ())),
                             preferred_element_type=jnp.float32)
        u_ref[...] = xw * dinv
        dinv_ref[...] = dinv

    return pl.pallas_call(
        body,
        out_shape=[jax.ShapeDtypeStruct((h, n), jnp.float32),
                   jax.ShapeDtypeStruct((1, n), jnp.float32)],
    )(x, w1, degp)


def _tc_mid(sT, uT, dinv, b, g, be, wn):
    h = sT.shape[0]
    n = sT.shape[1]
    hn = wn.shape[1]

    def body(s_ref, u_ref, dinv_ref, b_ref, g_ref, be_ref, w_ref, o_ref):
        dv = dinv_ref[...]
        z = dv * (s_ref[...] + u_ref[...]) + b_ref[...]
        m = jnp.mean(z, axis=1, keepdims=True)
        zc = z - m
        v = jnp.mean(zc * zc, axis=1, keepdims=True)
        hh = zc * lax.rsqrt(v + EPS) * g_ref[...] + be_ref[...]
        hh = jnp.maximum(hh, 0.0)
        o_ref[...] = lax.dot_general(w_ref[...], hh, (((0,), (0,)), ((), ())),
                                     preferred_element_type=jnp.float32) * dv

    return pl.pallas_call(
        body,
        out_shape=jax.ShapeDtypeStruct((hn, n), jnp.float32),
    )(sT, uT, dinv, b, g, be, wn)


def _tc_final(s3p, u3, dinv, b3):
    n = u3.shape[1]

    def body(s_ref, u_ref, dinv_ref, b_ref, o_ref):
        s = jnp.sum(s_ref[...], axis=0, keepdims=True)
        o_ref[...] = dinv_ref[...] * (s + u_ref[...]) + b_ref[...]

    return pl.pallas_call(
        body,
        out_shape=jax.ShapeDtypeStruct((1, n), jnp.float32),
    )(s3p, u3, dinv, b3)


def kernel(x, edge_index, edge_attr, W1, b1, g1, be1, W2, b2, g2, be2, W3, b3):
    n = x.shape[0]
    e = edge_attr.shape[0]
    h = W1.shape[1]

    row = edge_index[0].astype(jnp.int32)
    col = edge_index[1].astype(jnp.int32)
    ew = edge_attr.astype(jnp.float32)

    degp = _make_deg(n, e)(col, ew)
    u1T, dinv = _tc_first(x, W1, degp)
    scat = _make_edge_scatter(n, e, h)
    s1T = scat(u1T.reshape(h * n), row, col, ew).reshape(h, n)
    u2T = _tc_mid(s1T, u1T, dinv, b1[:, None], g1[:, None], be1[:, None], W2)
    s2T = scat(u2T.reshape(h * n), row, col, ew).reshape(h, n)
    u3 = _tc_mid(s2T, u2T, dinv, b2[:, None], g2[:, None], be2[:, None], W3)
    s3p = _make_edge_scalar(n, e)(u3.reshape(n), row, col, ew)
    out = _tc_final(s3p, u3, dinv, b3[:, None])
    return out.reshape(n, 1)
